# baseline (speedup 1.0000x reference)
import functools

import jax
import jax.numpy as jnp
from jax import lax
from jax.experimental import pallas as pl
from jax.experimental.pallas import tpu as pltpu
from jax.experimental.pallas import tpu_sc as plsc

_VOCAB = 1000000
_EMB = 32
_B = 16384
_UF = 3
_IF = 4
_H1 = 64
_H2 = 32

_PACK = 4
_PW = _PACK * _EMB
_VB = 8192
_RBLK = 31
_CH = _RBLK * _VB

_NC = 2
_NS = 16
_NW = _NC * _NS

_U_TOT = _B * _UF
_I_TOT = _B * _IF
_U_PW = _U_TOT // _NW
_I_PW = _I_TOT // _NW
_CHUNK = 128


def _repack_body(e0_ref, e1_ref, e2_ref, e3_ref, out_ref):
    stacked = jnp.concatenate(
        [e0_ref[...], e1_ref[...], e2_ref[...], e3_ref[...]], axis=0)
    out_ref[...] = stacked.T


_LASTVB = (_VOCAB + _VB - 1) // _VB - 1


def _repack(embT):
    def qspec(q):
        return pl.BlockSpec(
            (_EMB, _VB),
            lambda i, q=q: (0, jnp.minimum(q * _RBLK + i, _LASTVB)))

    return pl.pallas_call(
        _repack_body,
        grid=(_RBLK,),
        in_specs=[qspec(0), qspec(1), qspec(2), qspec(3)],
        out_specs=pl.BlockSpec((_VB, _PW), lambda i: (i, 0)),
        out_shape=jax.ShapeDtypeStruct((_CH, _PW), jnp.float32),
    )(embT, embT, embT, embT)


def _make_sc_gather(tot, per_w):
    def body(table_hbm, idx_hbm, out_hbm, idx_v, rows_v, sem):
        wid = lax.axis_index("s") * _NC + lax.axis_index("c")
        base = wid * per_w
        pltpu.sync_copy(idx_hbm.at[pl.ds(base, per_w)], idx_v)
        copies = []
        for j in range(per_w // _CHUNK):
            sl = pl.ds(j * _CHUNK, _CHUNK)
            copies.append(pltpu.async_copy(
                table_hbm.at[idx_v.at[sl]], rows_v.at[sl], sem))
        for c in copies:
            c.wait()
        pltpu.sync_copy(rows_v,
                        out_hbm.at[pl.ds(base, per_w), pl.ds(0, _EMB)])

    return functools.partial(
        pl.kernel,
        out_type=jax.ShapeDtypeStruct((tot, _PW), jnp.float32),
        mesh=plsc.VectorSubcoreMesh(core_axis_name="c", subcore_axis_name="s"),
        compiler_params=pltpu.CompilerParams(use_tc_tiling_on_sc=False),
        scratch_types=[
            pltpu.VMEM((per_w,), jnp.int32),
            pltpu.VMEM((per_w, _EMB), jnp.float32),
            pltpu.SemaphoreType.DMA,
        ],
    )(body)


_sc_gather_u = _make_sc_gather(_U_TOT, _U_PW)
_sc_gather_i = _make_sc_gather(_I_TOT, _I_PW)


_BLK = 4096
_UBPF = _B // _BLK


def _norm_t(v, ones, hi):
    s = jnp.dot(v * v, ones, precision=hi,
                preferred_element_type=jnp.float32)
    return (v * jax.lax.rsqrt(jnp.maximum(s, 1e-24))).T


def _dot_t(x, wt, hi):
    return jax.lax.dot_general(
        x, wt, (((1,), (1,)), ((), ())), precision=hi,
        preferred_element_type=jnp.float32)


def _tower_u_body(u0_ref, u1_ref, u2_ref, wu1_ref, bu1_ref, wu2_ref, bu2_ref,
                  uv_ref):
    hi = jax.lax.Precision.DEFAULT
    uc = jnp.concatenate(
        [u0_ref[:, :_EMB], u1_ref[:, :_EMB], u2_ref[:, :_EMB]], axis=1)
    uh = jnp.maximum(_dot_t(uc, wu1_ref[...], hi) + bu1_ref[...], 0.0)
    uv = jnp.maximum(_dot_t(uh, wu2_ref[...], hi) + bu2_ref[...], 0.0)
    uv_ref[...] = _norm_t(uv, jnp.ones((_H2, _H2), jnp.float32), hi)


def _tower_i_body(i0_ref, i1_ref, i2_ref, i3_ref,
                  wi1_ref, bi1_ref, wi2_ref, bi2_ref, uvt_ref,
                  score_ref, iv_ref):
    hi = jax.lax.Precision.DEFAULT
    ic = jnp.concatenate(
        [i0_ref[:, :_EMB], i1_ref[:, :_EMB], i2_ref[:, :_EMB],
         i3_ref[:, :_EMB]], axis=1)
    ih = jnp.maximum(_dot_t(ic, wi1_ref[...], hi) + bi1_ref[...], 0.0)
    iv = jnp.maximum(_dot_t(ih, wi2_ref[...], hi) + bi2_ref[...], 0.0)
    ivt = _norm_t(iv, jnp.ones((_H2, _H2), jnp.float32), hi)
    iv_ref[...] = ivt
    score_ref[...] = jnp.sum(uvt_ref[...] * ivt, axis=0)


def _full(shape):
    return pl.BlockSpec(shape, lambda i: (0,) * len(shape))


def _fspec(f):
    return pl.BlockSpec((_BLK, _PW), lambda i, f=f: (f * _UBPF + i, 0))


def _tower_u(urows, Wu1, bu1, Wu2, bu2):
    return pl.pallas_call(
        _tower_u_body,
        grid=(_B // _BLK,),
        in_specs=[
            _fspec(0), _fspec(1), _fspec(2),
            _full((_H1, _UF * _EMB)),
            _full((1, _H1)),
            _full((_H2, _H1)),
            _full((1, _H2)),
        ],
        out_specs=pl.BlockSpec((_H2, _BLK), lambda i: (0, i)),
        out_shape=jax.ShapeDtypeStruct((_H2, _B), jnp.float32),
    )(urows, urows, urows,
      Wu1.T, bu1.reshape(1, _H1), Wu2.T, bu2.reshape(1, _H2))


def _tower_i(irows, uvt, Wi1, bi1, Wi2, bi2):
    return pl.pallas_call(
        _tower_i_body,
        grid=(_B // _BLK,),
        in_specs=[
            _fspec(0), _fspec(1), _fspec(2), _fspec(3),
            _full((_H1, _IF * _EMB)),
            _full((1, _H1)),
            _full((_H2, _H1)),
            _full((1, _H2)),
            pl.BlockSpec((_H2, _BLK), lambda i: (0, i)),
        ],
        out_specs=[
            pl.BlockSpec((_BLK,), lambda i: (i,)),
            pl.BlockSpec((_H2, _BLK), lambda i: (0, i)),
        ],
        out_shape=[
            jax.ShapeDtypeStruct((_B,), jnp.float32),
            jax.ShapeDtypeStruct((_H2, _B), jnp.float32),
        ],
    )(irows, irows, irows, irows,
      Wi1.T, bi1.reshape(1, _H1), Wi2.T, bi2.reshape(1, _H2), uvt)


def kernel(user_inputs, item_inputs, emb, Wu1, bu1, Wu2, bu2, Wi1, bi1, Wi2, bi2):
    user_inputs = user_inputs.astype(jnp.int32)
    item_inputs = item_inputs.astype(jnp.int32)
    packed = _repack(emb.T)
    table = packed.reshape(_PACK * _CH, _EMB)
    ur = (_PACK * (user_inputs % _CH) + user_inputs // _CH).T.reshape(-1)
    ir = (_PACK * (item_inputs % _CH) + item_inputs // _CH).T.reshape(-1)
    urows = _sc_gather_u(table, ur)
    irows = _sc_gather_i(table, ir)
    uvt = _tower_u(urows, Wu1, bu1, Wu2, bu2)
    score, ivt = _tower_i(irows, uvt, Wi1, bi1, Wi2, bi2)
    return (score, uvt.T, ivt.T)


# speedup vs baseline: 1.0058x; 1.0058x over previous
import functools

import jax
import jax.numpy as jnp
from jax import lax
from jax.experimental import pallas as pl
from jax.experimental.pallas import tpu as pltpu
from jax.experimental.pallas import tpu_sc as plsc

_VOCAB = 1000000
_EMB = 32
_B = 16384
_UF = 3
_IF = 4
_H1 = 64
_H2 = 32

_PACK = 4
_PW = _PACK * _EMB
_VB = 8192
_RBLK = 31
_CH = _RBLK * _VB

_NC = 2
_NS = 16
_NW = _NC * _NS

_U_TOT = _B * _UF
_I_TOT = _B * _IF
_U_PW = _U_TOT // _NW
_I_PW = _I_TOT // _NW
_CHUNK = 128


def _repack_body(e0_ref, e1_ref, e2_ref, e3_ref, out_ref):
    stacked = jnp.concatenate(
        [e0_ref[...], e1_ref[...], e2_ref[...], e3_ref[...]], axis=0)
    out_ref[...] = stacked.T


_LASTVB = (_VOCAB + _VB - 1) // _VB - 1


def _repack(embT):
    def qspec(q):
        return pl.BlockSpec(
            (_EMB, _VB),
            lambda i, q=q: (0, jnp.minimum(q * _RBLK + i, _LASTVB)))

    return pl.pallas_call(
        _repack_body,
        grid=(_RBLK,),
        in_specs=[qspec(0), qspec(1), qspec(2), qspec(3)],
        out_specs=pl.BlockSpec((_VB, _PW), lambda i: (i, 0)),
        out_shape=jax.ShapeDtypeStruct((_CH, _PW), jnp.float32),
    )(embT, embT, embT, embT)


def _make_sc_gather(tot, per_w):
    def body(table_hbm, idx_hbm, out_hbm, idx_v, rows_v, sem):
        wid = lax.axis_index("s") * _NC + lax.axis_index("c")
        base = wid * per_w
        pltpu.sync_copy(idx_hbm.at[pl.ds(base, per_w)], idx_v)
        copies = []
        for j in range(per_w // _CHUNK):
            sl = pl.ds(j * _CHUNK, _CHUNK)
            copies.append(pltpu.async_copy(
                table_hbm.at[idx_v.at[sl]], rows_v.at[sl], sem))
        for c in copies:
            c.wait()
        pltpu.sync_copy(rows_v,
                        out_hbm.at[pl.ds(base, per_w), pl.ds(0, _EMB)])

    return functools.partial(
        pl.kernel,
        out_type=jax.ShapeDtypeStruct((tot, _PW), jnp.float32),
        mesh=plsc.VectorSubcoreMesh(core_axis_name="c", subcore_axis_name="s"),
        compiler_params=pltpu.CompilerParams(use_tc_tiling_on_sc=False),
        scratch_types=[
            pltpu.VMEM((per_w,), jnp.int32),
            pltpu.VMEM((per_w, _EMB), jnp.float32),
            pltpu.SemaphoreType.DMA,
        ],
    )(body)


_sc_gather_u = _make_sc_gather(_U_TOT, _U_PW)
_sc_gather_i = _make_sc_gather(_I_TOT, _I_PW)


_BLK = 8192
_UBPF = _B // _BLK


def _norm_t(v, ones, hi):
    s = jnp.dot(v * v, ones, precision=hi,
                preferred_element_type=jnp.float32)
    return (v * jax.lax.rsqrt(jnp.maximum(s, 1e-24))).T


def _dot_t(x, wt, hi):
    return jax.lax.dot_general(
        x, wt, (((1,), (1,)), ((), ())), precision=hi,
        preferred_element_type=jnp.float32)


def _tower_u_body(u0_ref, u1_ref, u2_ref, wu1_ref, bu1_ref, wu2_ref, bu2_ref,
                  uv_ref):
    hi = jax.lax.Precision.DEFAULT
    uc = jnp.concatenate(
        [u0_ref[:, :_EMB], u1_ref[:, :_EMB], u2_ref[:, :_EMB]], axis=1)
    uh = jnp.maximum(_dot_t(uc, wu1_ref[...], hi) + bu1_ref[...], 0.0)
    uv = jnp.maximum(_dot_t(uh, wu2_ref[...], hi) + bu2_ref[...], 0.0)
    uv_ref[...] = _norm_t(uv, jnp.ones((_H2, _H2), jnp.float32), hi)


def _tower_i_body(i0_ref, i1_ref, i2_ref, i3_ref,
                  wi1_ref, bi1_ref, wi2_ref, bi2_ref, uvt_ref,
                  score_ref, iv_ref):
    hi = jax.lax.Precision.DEFAULT
    ic = jnp.concatenate(
        [i0_ref[:, :_EMB], i1_ref[:, :_EMB], i2_ref[:, :_EMB],
         i3_ref[:, :_EMB]], axis=1)
    ih = jnp.maximum(_dot_t(ic, wi1_ref[...], hi) + bi1_ref[...], 0.0)
    iv = jnp.maximum(_dot_t(ih, wi2_ref[...], hi) + bi2_ref[...], 0.0)
    ivt = _norm_t(iv, jnp.ones((_H2, _H2), jnp.float32), hi)
    iv_ref[...] = ivt
    score_ref[...] = jnp.sum(uvt_ref[...] * ivt, axis=0)


def _full(shape):
    return pl.BlockSpec(shape, lambda i: (0,) * len(shape))


def _fspec(f):
    return pl.BlockSpec((_BLK, _PW), lambda i, f=f: (f * _UBPF + i, 0))


def _tower_u(urows, Wu1, bu1, Wu2, bu2):
    return pl.pallas_call(
        _tower_u_body,
        grid=(_B // _BLK,),
        in_specs=[
            _fspec(0), _fspec(1), _fspec(2),
            _full((_H1, _UF * _EMB)),
            _full((1, _H1)),
            _full((_H2, _H1)),
            _full((1, _H2)),
        ],
        out_specs=pl.BlockSpec((_H2, _BLK), lambda i: (0, i)),
        out_shape=jax.ShapeDtypeStruct((_H2, _B), jnp.float32),
    )(urows, urows, urows,
      Wu1.T, bu1.reshape(1, _H1), Wu2.T, bu2.reshape(1, _H2))


def _tower_i(irows, uvt, Wi1, bi1, Wi2, bi2):
    return pl.pallas_call(
        _tower_i_body,
        grid=(_B // _BLK,),
        in_specs=[
            _fspec(0), _fspec(1), _fspec(2), _fspec(3),
            _full((_H1, _IF * _EMB)),
            _full((1, _H1)),
            _full((_H2, _H1)),
            _full((1, _H2)),
            pl.BlockSpec((_H2, _BLK), lambda i: (0, i)),
        ],
        out_specs=[
            pl.BlockSpec((_BLK,), lambda i: (i,)),
            pl.BlockSpec((_H2, _BLK), lambda i: (0, i)),
        ],
        out_shape=[
            jax.ShapeDtypeStruct((_B,), jnp.float32),
            jax.ShapeDtypeStruct((_H2, _B), jnp.float32),
        ],
    )(irows, irows, irows, irows,
      Wi1.T, bi1.reshape(1, _H1), Wi2.T, bi2.reshape(1, _H2), uvt)


def kernel(user_inputs, item_inputs, emb, Wu1, bu1, Wu2, bu2, Wi1, bi1, Wi2, bi2):
    user_inputs = user_inputs.astype(jnp.int32)
    item_inputs = item_inputs.astype(jnp.int32)
    packed = _repack(emb.T)
    table = packed.reshape(_PACK * _CH, _EMB)
    ur = (_PACK * (user_inputs % _CH) + user_inputs // _CH).T.reshape(-1)
    ir = (_PACK * (item_inputs % _CH) + item_inputs // _CH).T.reshape(-1)
    urows = _sc_gather_u(table, ur)
    irows = _sc_gather_i(table, ir)
    uvt = _tower_u(urows, Wu1, bu1, Wu2, bu2)
    score, ivt = _tower_i(irows, uvt, Wi1, bi1, Wi2, bi2)
    return (score, uvt.T, ivt.T)


# speedup vs baseline: 1.0148x; 1.0089x over previous
import functools

import jax
import jax.numpy as jnp
from jax import lax
from jax.experimental import pallas as pl
from jax.experimental.pallas import tpu as pltpu
from jax.experimental.pallas import tpu_sc as plsc

_VOCAB = 1000000
_EMB = 32
_B = 16384
_UF = 3
_IF = 4
_H1 = 64
_H2 = 32

_PACK = 4
_PW = _PACK * _EMB
_VB = 16384
_RBLK = 16
_CH = _RBLK * _VB

_NC = 2
_NS = 16
_NW = _NC * _NS

_U_TOT = _B * _UF
_I_TOT = _B * _IF
_U_PW = _U_TOT // _NW
_I_PW = _I_TOT // _NW
_CHUNK = 128


def _repack_body(e0_ref, e1_ref, e2_ref, e3_ref, out_ref):
    stacked = jnp.concatenate(
        [e0_ref[...], e1_ref[...], e2_ref[...], e3_ref[...]], axis=0)
    out_ref[...] = stacked.T


_LASTVB = (_VOCAB + _VB - 1) // _VB - 1


def _repack(embT):
    def qspec(q):
        return pl.BlockSpec(
            (_EMB, _VB),
            lambda i, q=q: (0, jnp.minimum(q * _RBLK + i, _LASTVB)))

    return pl.pallas_call(
        _repack_body,
        grid=(_RBLK,),
        in_specs=[qspec(0), qspec(1), qspec(2), qspec(3)],
        out_specs=pl.BlockSpec((_VB, _PW), lambda i: (i, 0)),
        out_shape=jax.ShapeDtypeStruct((_CH, _PW), jnp.float32),
    )(embT, embT, embT, embT)


def _make_sc_gather(tot, per_w):
    def body(table_hbm, idx_hbm, out_hbm, idx_v, rows_v, sem):
        wid = lax.axis_index("s") * _NC + lax.axis_index("c")
        base = wid * per_w
        pltpu.sync_copy(idx_hbm.at[pl.ds(base, per_w)], idx_v)
        copies = []
        for j in range(per_w // _CHUNK):
            sl = pl.ds(j * _CHUNK, _CHUNK)
            copies.append(pltpu.async_copy(
                table_hbm.at[idx_v.at[sl]], rows_v.at[sl], sem))
        for c in copies:
            c.wait()
        pltpu.sync_copy(rows_v,
                        out_hbm.at[pl.ds(base, per_w), pl.ds(0, _EMB)])

    return functools.partial(
        pl.kernel,
        out_type=jax.ShapeDtypeStruct((tot, _PW), jnp.float32),
        mesh=plsc.VectorSubcoreMesh(core_axis_name="c", subcore_axis_name="s"),
        compiler_params=pltpu.CompilerParams(use_tc_tiling_on_sc=False),
        scratch_types=[
            pltpu.VMEM((per_w,), jnp.int32),
            pltpu.VMEM((per_w, _EMB), jnp.float32),
            pltpu.SemaphoreType.DMA,
        ],
    )(body)


_sc_gather_u = _make_sc_gather(_U_TOT, _U_PW)
_sc_gather_i = _make_sc_gather(_I_TOT, _I_PW)


_BLK = 8192
_UBPF = _B // _BLK


def _norm_t(v, ones, hi):
    s = jnp.dot(v * v, ones, precision=hi,
                preferred_element_type=jnp.float32)
    return (v * jax.lax.rsqrt(jnp.maximum(s, 1e-24))).T


def _dot_t(x, wt, hi):
    return jax.lax.dot_general(
        x, wt, (((1,), (1,)), ((), ())), precision=hi,
        preferred_element_type=jnp.float32)


def _tower_u_body(u0_ref, u1_ref, u2_ref, wu1_ref, bu1_ref, wu2_ref, bu2_ref,
                  uv_ref):
    hi = jax.lax.Precision.DEFAULT
    uc = jnp.concatenate(
        [u0_ref[:, :_EMB], u1_ref[:, :_EMB], u2_ref[:, :_EMB]], axis=1)
    uh = jnp.maximum(_dot_t(uc, wu1_ref[...], hi) + bu1_ref[...], 0.0)
    uv = jnp.maximum(_dot_t(uh, wu2_ref[...], hi) + bu2_ref[...], 0.0)
    uv_ref[...] = _norm_t(uv, jnp.ones((_H2, _H2), jnp.float32), hi)


def _tower_i_body(i0_ref, i1_ref, i2_ref, i3_ref,
                  wi1_ref, bi1_ref, wi2_ref, bi2_ref, uvt_ref,
                  score_ref, iv_ref):
    hi = jax.lax.Precision.DEFAULT
    ic = jnp.concatenate(
        [i0_ref[:, :_EMB], i1_ref[:, :_EMB], i2_ref[:, :_EMB],
         i3_ref[:, :_EMB]], axis=1)
    ih = jnp.maximum(_dot_t(ic, wi1_ref[...], hi) + bi1_ref[...], 0.0)
    iv = jnp.maximum(_dot_t(ih, wi2_ref[...], hi) + bi2_ref[...], 0.0)
    ivt = _norm_t(iv, jnp.ones((_H2, _H2), jnp.float32), hi)
    iv_ref[...] = ivt
    score_ref[...] = jnp.sum(uvt_ref[...] * ivt, axis=0)


def _full(shape):
    return pl.BlockSpec(shape, lambda i: (0,) * len(shape))


def _fspec(f):
    return pl.BlockSpec((_BLK, _PW), lambda i, f=f: (f * _UBPF + i, 0))


def _tower_u(urows, Wu1, bu1, Wu2, bu2):
    return pl.pallas_call(
        _tower_u_body,
        grid=(_B // _BLK,),
        in_specs=[
            _fspec(0), _fspec(1), _fspec(2),
            _full((_H1, _UF * _EMB)),
            _full((1, _H1)),
            _full((_H2, _H1)),
            _full((1, _H2)),
        ],
        out_specs=pl.BlockSpec((_H2, _BLK), lambda i: (0, i)),
        out_shape=jax.ShapeDtypeStruct((_H2, _B), jnp.float32),
    )(urows, urows, urows,
      Wu1.T, bu1.reshape(1, _H1), Wu2.T, bu2.reshape(1, _H2))


def _tower_i(irows, uvt, Wi1, bi1, Wi2, bi2):
    return pl.pallas_call(
        _tower_i_body,
        grid=(_B // _BLK,),
        in_specs=[
            _fspec(0), _fspec(1), _fspec(2), _fspec(3),
            _full((_H1, _IF * _EMB)),
            _full((1, _H1)),
            _full((_H2, _H1)),
            _full((1, _H2)),
            pl.BlockSpec((_H2, _BLK), lambda i: (0, i)),
        ],
        out_specs=[
            pl.BlockSpec((_BLK,), lambda i: (i,)),
            pl.BlockSpec((_H2, _BLK), lambda i: (0, i)),
        ],
        out_shape=[
            jax.ShapeDtypeStruct((_B,), jnp.float32),
            jax.ShapeDtypeStruct((_H2, _B), jnp.float32),
        ],
    )(irows, irows, irows, irows,
      Wi1.T, bi1.reshape(1, _H1), Wi2.T, bi2.reshape(1, _H2), uvt)


def kernel(user_inputs, item_inputs, emb, Wu1, bu1, Wu2, bu2, Wi1, bi1, Wi2, bi2):
    user_inputs = user_inputs.astype(jnp.int32)
    item_inputs = item_inputs.astype(jnp.int32)
    packed = _repack(emb.T)
    table = packed.reshape(_PACK * _CH, _EMB)
    ur = (_PACK * (user_inputs % _CH) + user_inputs // _CH).T.reshape(-1)
    ir = (_PACK * (item_inputs % _CH) + item_inputs // _CH).T.reshape(-1)
    urows = _sc_gather_u(table, ur)
    irows = _sc_gather_i(table, ir)
    uvt = _tower_u(urows, Wu1, bu1, Wu2, bu2)
    score, ivt = _tower_i(irows, uvt, Wi1, bi1, Wi2, bi2)
    return (score, uvt.T, ivt.T)
